# baseline (speedup 1.0000x reference)
import functools

import jax
import jax.numpy as jnp
from jax.experimental import pallas as pl

N = 10000; E = 160000; T = 320000; B = 64
H = 128; OUT = 128; INT = 64; OEMB = 256; NS = 7; NR = 6; BE = 8
NODE_DIM = 128; EDGE_DIM = 16; FP = 2048; CUTOFF = 5.0; PEXP = 6; NZ = 95

EB = 2000
TB = 1600
NB = 2000

F32 = jnp.float32


HP = jax.lax.Precision.HIGHEST


def _dot(a, b):
    return jnp.dot(a, b)


def _swish(t):
    return t * jax.nn.sigmoid(t)


def _full(x):
    return pl.BlockSpec(x.shape, lambda i: (0,) * x.ndim)


def _rows(bs, x):
    return pl.BlockSpec((bs,) + x.shape[1:], lambda i: (i,) + (0,) * (x.ndim - 1))


def _call_rows(body, bs, nrows, row_args, bcast_args, out_width):
    grid = (nrows // bs,)
    in_specs = [_rows(bs, a) for a in row_args] + [_full(a) for a in bcast_args]
    return pl.pallas_call(
        body,
        grid=grid,
        in_specs=in_specs,
        out_specs=pl.BlockSpec((bs, out_width), lambda i: (i, 0)),
        out_shape=jax.ShapeDtypeStruct((nrows, out_width), F32),
    )(*row_args, *bcast_args)


def _init_e_body(hd, hs, rbf, w0, w1, wr0, wr1, b, o):
    acc = (_dot(hd[...], w0[...]) + _dot(hs[...], w1[...])
           + _dot(_dot(rbf[...], wr0[...]), wr1[...]) + b[...])
    o[...] = _swish(acc)


def _edge_pre_body(e, rbf, wji, bji, wkj, bkj, wr0, wr1, down, xji_o, xkd_o):
    ev = e[...]
    xji_o[...] = _swish(_dot(ev, wji[...]) + bji[...])
    xk = _swish(_dot(ev, wkj[...]) + bkj[...]) * _dot(_dot(rbf[...], wr0[...]), wr1[...])
    xkd_o[...] = _swish(_dot(xk, down[...]))


def _edge_pre(e, rbf, wji, bji, wkj, bkj, wr0, wr1, down):
    grid = (E // EB,)
    row_args = [e, rbf]
    bcast = [wji, bji, wkj, bkj, wr0, wr1, down]
    return pl.pallas_call(
        _edge_pre_body,
        grid=grid,
        in_specs=[_rows(EB, a) for a in row_args] + [_full(a) for a in bcast],
        out_specs=[pl.BlockSpec((EB, H), lambda i: (i, 0)),
                   pl.BlockSpec((EB, INT), lambda i: (i, 0))],
        out_shape=[jax.ShapeDtypeStruct((E, H), F32),
                   jax.ShapeDtypeStruct((E, INT), F32)],
    )(*row_args, *bcast)


def _trip_body(g, cbf, tcf, rad, s1, s2, t1, t2, o):
    radv = rad[...]
    cbfv = cbf[...]
    tcfv = tcf[...]
    s1v = s1[...]
    t1v = t1[...]
    m8s = _dot(cbfv[:, 0:1] * radv, s1v[0:NR])
    for a in range(1, NS):
        m8s += _dot(cbfv[:, a:a + 1] * radv, s1v[a * NR:(a + 1) * NR])
    smod = _dot(m8s, s2[...])
    xcat = jnp.concatenate([tcfv[:, b:b + 1] * radv for b in range(NS)], axis=1)
    m8t = _dot(cbfv[:, 0:1] * xcat, t1v[0:NS * NR])
    for a in range(1, NS):
        m8t += _dot(cbfv[:, a:a + 1] * xcat, t1v[a * NS * NR:(a + 1) * NS * NR])
    tmod = _dot(m8t, t2[...])
    o[...] = g[...] * smod * tmod


def _trip(g, cbf, tcf, rad, s1, s2, t1, t2):
    return _call_rows(_trip_body, TB, T, [g, cbf, tcf, rad], [s1, s2, t1, t2], INT)


def _edge_post_body(agg, xji, e, up, r0w1, r0b1, r0w2, r0b2, lin, blin,
                    r1w1, r1b1, r1w2, r1b2, r2w1, r2b1, r2w2, r2b2, o):
    e2 = xji[...] + _swish(_dot(agg[...], up[...]))
    e2 = e2 + _swish(_dot(_swish(_dot(e2, r0w1[...]) + r0b1[...]), r0w2[...]) + r0b2[...])
    e2 = _swish(_dot(e2, lin[...]) + blin[...]) + e[...]
    e2 = e2 + _swish(_dot(_swish(_dot(e2, r1w1[...]) + r1b1[...]), r1w2[...]) + r1b2[...])
    e2 = e2 + _swish(_dot(_swish(_dot(e2, r2w1[...]) + r2b1[...]), r2w2[...]) + r2b2[...])
    o[...] = e2


def _edge_post(agg, xji, e, ep):
    (r0w1, r0b1, r0w2, r0b2), = ep['res_before']
    (r1w1, r1b1, r1w2, r1b2), (r2w1, r2b1, r2w2, r2b2) = ep['res_after']
    bcast = [ep['up'], r0w1, r0b1.reshape(1, -1), r0w2, r0b2.reshape(1, -1),
             ep['lin'], ep['b_lin'].reshape(1, -1),
             r1w1, r1b1.reshape(1, -1), r1w2, r1b2.reshape(1, -1),
             r2w1, r2b1.reshape(1, -1), r2w2, r2b2.reshape(1, -1)]
    return _call_rows(_edge_post_body, EB, E, [agg, xji, e], bcast, H)


def _node_body(vagg, up, w0, b0, w1, b1, w2, b2, wo, o):
    v = _dot(vagg[...], up[...])
    v = _swish(_dot(v, w0[...]) + b0[...])
    v = _swish(_dot(v, w1[...]) + b1[...])
    v = _swish(_dot(v, w2[...]) + b2[...])
    o[...] = _dot(v, wo[...])


def _node(vagg, vp):
    (w0, b0), (w1, b1), (w2, b2) = vp['layers']
    bcast = [vp['up'], w0, b0.reshape(1, -1), w1, b1.reshape(1, -1),
             w2, b2.reshape(1, -1), vp['out']]
    return _call_rows(_node_body, NB, N, [vagg], bcast, OUT)


def _gine_msg_body(hs, ea, we, be, o):
    o[...] = jnp.maximum(hs[...] + _dot(ea[...], we[...]) + be[...], 0.0)


def _gine_msg(hs, ea, we, be):
    return _call_rows(_gine_msg_body, EB, E, [hs, ea], [we, be.reshape(1, -1)], H)


def _gine_node_body(h, agg, nnw, nnb, eps, o):
    ov = (1.0 + eps[0, 0]) * h[...] + agg[...]
    o[...] = jnp.maximum(_dot(ov, nnw[...]) + nnb[...], 0.0)


def _gine_node(h, agg, nnw, nnb, eps):
    return _call_rows(_gine_node_body, NB, N, [h, agg],
                      [nnw, nnb.reshape(1, -1), eps.reshape(1, 1)], H)


def _fcn_body(fp, w1, b1, w2, b2, o):
    o[...] = _dot(jnp.maximum(_dot(fp[...], w1[...]) + b1[...], 0.0), w2[...]) + b2[...]


def _head_body(cat, hw, hb, ow, ob, o):
    h = jnp.maximum(_dot(cat[...], hw[...]) + hb[...], 0.0)
    o[...] = _dot(h, ow[...]) + ob[...]


def _one_block(body, args, out_shape):
    return pl.pallas_call(
        body,
        in_specs=[pl.BlockSpec(a.shape, lambda: (0,) * a.ndim) for a in args],
        out_specs=pl.BlockSpec(out_shape, lambda: (0, 0)),
        out_shape=jax.ShapeDtypeStruct(out_shape, F32),
    )(*args)


def kernel(z, pos, batch, edge_index, idx_kj, idx_ji, x, edge_attr, fingerprints, params):
    p = params
    src = edge_index[0]; dst = edge_index[1]
    vec = pos[dst] - pos[src]
    dist = jnp.sqrt(jnp.sum(vec * vec, -1) + 1e-12)
    pp = float(PEXP)
    a_ = -(pp + 1) * (pp + 2) / 2.0; b_ = pp * (pp + 2); c_ = -pp * (pp + 1) / 2.0

    def envelope(xx):
        env = 1.0 / xx + a_ * xx ** (PEXP - 1) + b_ * xx ** PEXP + c_ * xx ** (PEXP + 1)
        return jnp.where(xx < 1.0, env, 0.0)

    freqs = jnp.arange(1, NR + 1, dtype=F32) * jnp.pi
    xr = jnp.clip(dist / CUTOFF, 1e-3, None)
    rbf = envelope(xr)[:, None] * jnp.sin(freqs[None, :] * xr[:, None])
    va = vec[idx_ji]; vb = vec[idx_kj]
    cr = jnp.cross(va, vb)
    angle = jnp.arctan2(jnp.sqrt(jnp.sum(cr * cr, -1)) + 1e-9, jnp.sum(va * vb, -1))
    torsion = jnp.arctan2(cr[:, 0] + 1e-9, cr[:, 1] + 1e-9)
    x_kjr = jnp.clip(dist[idx_kj] / CUTOFF, 1e-3, None)
    rad = envelope(x_kjr)[:, None] * jnp.sin(freqs[None, :] * x_kjr[:, None])
    ls = jnp.arange(NS, dtype=F32)
    cbf = jnp.cos(ls[None, :] * angle[:, None])
    tcf = jnp.cos(ls[None, :] * torsion[:, None])

    h = p['emb_z'][z]
    hd = h[dst]; hs = h[src]
    w0 = p['init_W'][:H]; w1 = p['init_W'][H:2 * H]; w2 = p['init_W'][2 * H:]
    ib = p['init_b'].reshape(1, -1)
    e = pl.pallas_call(
        _init_e_body,
        grid=(E // EB,),
        in_specs=[_rows(EB, a) for a in (hd, hs, rbf)] +
                 [_full(a) for a in (w0, w1, p['init_rbf'], w2, ib)],
        out_specs=pl.BlockSpec((EB, H), lambda i: (i, 0)),
        out_shape=jax.ShapeDtypeStruct((E, H), F32),
    )(hd, hs, rbf, w0, w1, p['init_rbf'], w2, ib)

    def update_e(e_, ep):
        xji, xkd = _edge_pre(e_, rbf, ep['W_ji'], ep['b_ji'].reshape(1, -1),
                             ep['W_kj'], ep['b_kj'].reshape(1, -1),
                             ep['rbf1'], ep['rbf2'], ep['down'])
        g = xkd[idx_kj]
        m = _trip(g, cbf, tcf, rad, ep['sbf1'], ep['sbf2'], ep['t1'], ep['t2'])
        agg = jax.ops.segment_sum(m, idx_ji, num_segments=E)
        return _edge_post(agg, xji, e_, ep)

    def update_v(e_, vp):
        vagg = jax.ops.segment_sum(e_, dst, num_segments=N)
        return _node(vagg, vp)

    def gine(h_, cp):
        msg = _gine_msg(h_[src], edge_attr, cp['We'], cp['be'])
        agg = jax.ops.segment_sum(msg, dst, num_segments=N)
        return _gine_node(h_, agg, p['nn_W'], p['nn_b'], cp['eps'])

    v = update_v(e, p['init_v'])
    u = jax.ops.segment_sum(v, batch, num_segments=B)
    for l in range(4):
        e = update_e(e, p['ue'][l])
        v = update_v(e, p['uv'][l])
        u = u + jax.ops.segment_sum(v, batch, num_segments=B)
    sphere_out = v; sphere_e = e; sphere_u = u

    hid = x @ p['n2h_W'] + p['n2h_b']
    for i in range(3):
        hid = jnp.maximum(gine(hid, p['convs'][i]), 0.0)
    gnn_out = hid
    fcn_out = _one_block(
        _fcn_body,
        [fingerprints, p['fp_W1'], p['fp_b1'].reshape(1, -1),
         p['fp_W2'], p['fp_b2'].reshape(1, -1)],
        (B, H))
    sphere_out = sphere_out + gnn_out
    gnn_out = sphere_out
    sphere_e = update_e(sphere_e, p['ue'][3])
    sphere_v = update_v(sphere_e, p['uv'][3])
    sphere_out = sphere_u + jax.ops.segment_sum(sphere_v, batch, num_segments=B)
    gnn_out = jnp.maximum(gine(gnn_out, p['convs'][3]), 0.0)
    gnn_out = jax.ops.segment_sum(gnn_out, batch, num_segments=B)
    cat = jnp.concatenate([gnn_out, fcn_out, sphere_out], axis=1)
    out = _one_block(
        _head_body,
        [cat, p['hid_W'], p['hid_b'].reshape(1, -1),
         p['out_W'], p['out_b'].reshape(1, -1)],
        (B, 1))
    return (out, gnn_out, sphere_out)
